# Initial kernel scaffold; baseline (speedup 1.0000x reference)
import jax
import jax.numpy as jnp
from jax.experimental import pallas as pl


def kernel(updates, mask):
    raise NotImplementedError("write your pallas kernel here")


# speedup vs baseline: 28.5614x; 28.5614x over previous
import functools

import jax
import jax.numpy as jnp
import numpy as np
from jax import lax
from jax.experimental import pallas as pl
from jax.experimental.pallas import tpu as pltpu
from jax.experimental.pallas import tpu_sc as plsc

_B, _H, _W, _C = 2, 192, 192, 96
_P = _H * _W
_R = (_H * 2) * (_W * 2)
_HALF = _R // 2
_CHUNK = 6144
_NPAIRS = _B * _C
_NWORK = 32
_TASKS_PER_W = _NPAIRS // _NWORK

_THIRD = np.float32(1.0 / 3.0)


def _sc_unpool(mask_t, upd_t):
    mesh = plsc.VectorSubcoreMesh(core_axis_name="c", subcore_axis_name="s")

    @functools.partial(
        pl.kernel,
        mesh=mesh,
        out_type=jax.ShapeDtypeStruct((_NPAIRS, _R), jnp.float32),
        scratch_types=[
            pltpu.VMEM((_HALF,), jnp.float32),
            pltpu.VMEM((_CHUNK,), jnp.int32),
            pltpu.VMEM((_CHUNK,), jnp.float32),
        ],
        compiler_params=pltpu.CompilerParams(needs_layout_passes=False),
    )
    def k(mask_hbm, upd_hbm, out_hbm, acc, mbuf, ubuf):
        wid = lax.axis_index("s") * 2 + lax.axis_index("c")

        def task(j, carry):
            pair = wid * _TASKS_PER_W + j

            for half in (0, 1):
                lo = half * _HALF

                zeros = jnp.zeros((16,), jnp.float32)

                def zero_body(i, c):
                    base = i * 128
                    for u in range(8):
                        acc[pl.ds(base + u * 16, 16)] = zeros
                    return c

                lax.fori_loop(0, _HALF // 128, zero_body, 0)

                def chunk_body(ck, c):
                    off = ck * _CHUNK
                    pltpu.sync_copy(mask_hbm.at[pair, pl.ds(off, _CHUNK)], mbuf)
                    pltpu.sync_copy(upd_hbm.at[pair, pl.ds(off, _CHUNK)], ubuf)

                    def grp(g, cc):
                        base = g * 64
                        for u in range(4):
                            s = base + u * 16
                            m = mbuf[pl.ds(s, 16)]
                            v = ubuf[pl.ds(s, 16)]
                            t5 = lax.shift_right_logical(m, 5)
                            r = (t5.astype(jnp.float32) * _THIRD).astype(
                                jnp.int32)
                            idx = r - lo
                            ok = (idx >= 0) & (idx < _HALF)
                            plsc.addupdate_scatter(acc, [idx], v, mask=ok)
                        return cc

                    lax.fori_loop(0, _CHUNK // 64, grp, 0)
                    return c

                lax.fori_loop(0, _P // _CHUNK, chunk_body, 0)
                pltpu.sync_copy(acc, out_hbm.at[pair, pl.ds(lo, _HALF)])

            return carry

        lax.fori_loop(0, _TASKS_PER_W, task, 0)

    return k(mask_t, upd_t)


def kernel(updates, mask):
    B, H, W, C = updates.shape
    Ho, Wo = H * 2, W * 2
    m32 = mask.astype(jnp.int32)
    u_t = updates.reshape(B, H * W, C).transpose(0, 2, 1).reshape(B * C, H * W)
    m_t = m32.reshape(B, H * W, C).transpose(0, 2, 1).reshape(B * C, H * W)
    out = _sc_unpool(m_t, u_t)
    return out.reshape(B, C, Ho * Wo).transpose(0, 2, 1).reshape(B, Ho, Wo, C)


# speedup vs baseline: 32.1098x; 1.1242x over previous
import functools

import jax
import jax.numpy as jnp
import numpy as np
from jax import lax
from jax.experimental import pallas as pl
from jax.experimental.pallas import tpu as pltpu
from jax.experimental.pallas import tpu_sc as plsc

_B, _H, _W, _C = 2, 192, 192, 96
_P = _H * _W
_R = (_H * 2) * (_W * 2)
_HALF = _R // 2
_CHUNK = 6144
_NPAIRS = _B * _C
_NWORK = 32
_TASKS_PER_W = _NPAIRS // _NWORK

_THIRD = np.float32(1.0 / 3.0)


def _sc_unpool(mask_t, upd_t):
    mesh = plsc.VectorSubcoreMesh(core_axis_name="c", subcore_axis_name="s")

    @functools.partial(
        pl.kernel,
        mesh=mesh,
        out_type=jax.ShapeDtypeStruct((_NPAIRS, _R), jnp.float32),
        scratch_types=[
            pltpu.VMEM((_HALF,), jnp.float32),
            pltpu.VMEM((2, _CHUNK), jnp.int32),
            pltpu.VMEM((2, _CHUNK), jnp.float32),
            pltpu.SemaphoreType.DMA,
            pltpu.SemaphoreType.DMA,
        ],
        compiler_params=pltpu.CompilerParams(needs_layout_passes=False),
    )
    def k(mask_hbm, upd_hbm, out_hbm, acc, mbuf, ubuf, sem_a, sem_b):
        wid = lax.axis_index("s") * 2 + lax.axis_index("c")
        sems = (sem_a, sem_b)
        nchunks = _P // _CHUNK

        def task(j, carry):
            pair = wid * _TASKS_PER_W + j

            def issue(ck, par):
                off = ck * _CHUNK
                cm = pltpu.async_copy(
                    mask_hbm.at[pair, pl.ds(off, _CHUNK)], mbuf.at[par],
                    sems[par])
                cu = pltpu.async_copy(
                    upd_hbm.at[pair, pl.ds(off, _CHUNK)], ubuf.at[par],
                    sems[par])
                return cm, cu

            for half in (0, 1):
                lo = half * _HALF

                zeros = jnp.zeros((16,), jnp.float32)

                def zero_body(i, c):
                    base = i * 128
                    for u in range(8):
                        acc[pl.ds(base + u * 16, 16)] = zeros
                    return c

                cps = {0: issue(0, 0)}
                lax.fori_loop(0, _HALF // 128, zero_body, 0)

                for ck in range(nchunks):
                    par = ck % 2
                    if ck + 1 < nchunks:
                        cps[ck + 1] = issue(ck + 1, (ck + 1) % 2)
                    for cp in cps.pop(ck):
                        cp.wait()

                    def grp(g, cc, par=par):
                        base = g * 64
                        for u in range(4):
                            s = base + u * 16
                            m = mbuf[par, pl.ds(s, 16)]
                            v = ubuf[par, pl.ds(s, 16)]
                            t5 = lax.shift_right_logical(m, 5)
                            r = (t5.astype(jnp.float32) * _THIRD).astype(
                                jnp.int32)
                            idx = r - lo
                            ok = (idx >= 0) & (idx < _HALF)
                            plsc.addupdate_scatter(acc, [idx], v, mask=ok)
                        return cc

                    lax.fori_loop(0, _CHUNK // 64, grp, 0)

                pltpu.sync_copy(acc, out_hbm.at[pair, pl.ds(lo, _HALF)])

            return carry

        lax.fori_loop(0, _TASKS_PER_W, task, 0)

    return k(mask_t, upd_t)


def kernel(updates, mask):
    B, H, W, C = updates.shape
    Ho, Wo = H * 2, W * 2
    m32 = mask.astype(jnp.int32)
    u_t = updates.reshape(B, H * W, C).transpose(0, 2, 1).reshape(B * C, H * W)
    m_t = m32.reshape(B, H * W, C).transpose(0, 2, 1).reshape(B * C, H * W)
    out = _sc_unpool(m_t, u_t)
    return out.reshape(B, C, Ho * Wo).transpose(0, 2, 1).reshape(B, Ho, Wo, C)


# speedup vs baseline: 62.4793x; 1.9458x over previous
import functools

import jax
import jax.numpy as jnp
import numpy as np
from jax import lax
from jax.experimental import pallas as pl
from jax.experimental.pallas import tpu as pltpu
from jax.experimental.pallas import tpu_sc as plsc

_B, _H, _W, _C = 2, 192, 192, 96
_P = _H * _W
_R = (_H * 2) * (_W * 2)
_HALF = _R // 2
_CHUNK = 6144
_NPAIRS = _B * _C
_NWORK = 32
_TASKS_PER_W = _NPAIRS // _NWORK

_THIRD = np.float32(1.0 / 3.0)


def _sc_unpool(mask_t, upd_t):
    mesh = plsc.VectorSubcoreMesh(core_axis_name="c", subcore_axis_name="s")

    @functools.partial(
        pl.kernel,
        mesh=mesh,
        out_type=jax.ShapeDtypeStruct((_NPAIRS, _R), jnp.float32),
        scratch_types=[
            pltpu.VMEM((_HALF,), jnp.float32),
            pltpu.VMEM((2, _CHUNK), jnp.int32),
            pltpu.VMEM((2, _CHUNK), jnp.float32),
            pltpu.SemaphoreType.DMA,
            pltpu.SemaphoreType.DMA,
        ],
        compiler_params=pltpu.CompilerParams(needs_layout_passes=False),
    )
    def k(mask_hbm, upd_hbm, out_hbm, acc, mbuf, ubuf, sem_a, sem_b):
        wid = lax.axis_index("s") * 2 + lax.axis_index("c")
        sems = (sem_a, sem_b)
        nchunks = _P // _CHUNK

        def task(j, carry):
            pair = wid * _TASKS_PER_W + j

            def issue(ck, par):
                off = ck * _CHUNK
                cm = pltpu.async_copy(
                    mask_hbm.at[pair, pl.ds(off, _CHUNK)], mbuf.at[par],
                    sems[par])
                cu = pltpu.async_copy(
                    upd_hbm.at[pair, pl.ds(off, _CHUNK)], ubuf.at[par],
                    sems[par])
                return cm, cu

            for half in (0, 1):
                lo = half * _HALF

                zeros = jnp.zeros((16,), jnp.float32)
                cps = {0: issue(0, 0)}

                @plsc.parallel_loop(0, _HALF // 16, unroll=8)
                def _(i):
                    acc[pl.ds(i * 16, 16)] = zeros

                for ck in range(nchunks):
                    par = ck % 2
                    if ck + 1 < nchunks:
                        cps[ck + 1] = issue(ck + 1, (ck + 1) % 2)
                    for cp in cps.pop(ck):
                        cp.wait()

                    @plsc.parallel_loop(0, _CHUNK // 16, unroll=8)
                    def _(g, par=par, lo=lo):
                        s = g * 16
                        m = mbuf[par, pl.ds(s, 16)]
                        v = ubuf[par, pl.ds(s, 16)]
                        t5 = lax.shift_right_logical(m, 5)
                        r = (t5.astype(jnp.float32) * _THIRD).astype(
                            jnp.int32)
                        idx = r - lo
                        ok = plsc.bitcast(idx, jnp.uint32) < jnp.uint32(_HALF)
                        plsc.addupdate_scatter(acc, [idx], v, mask=ok)

                pltpu.sync_copy(acc, out_hbm.at[pair, pl.ds(lo, _HALF)])

            return carry

        lax.fori_loop(0, _TASKS_PER_W, task, 0)

    return k(mask_t, upd_t)


def kernel(updates, mask):
    B, H, W, C = updates.shape
    Ho, Wo = H * 2, W * 2
    m32 = mask.astype(jnp.int32)
    u_t = updates.reshape(B, H * W, C).transpose(0, 2, 1).reshape(B * C, H * W)
    m_t = m32.reshape(B, H * W, C).transpose(0, 2, 1).reshape(B * C, H * W)
    out = _sc_unpool(m_t, u_t)
    return out.reshape(B, C, Ho * Wo).transpose(0, 2, 1).reshape(B, Ho, Wo, C)


# speedup vs baseline: 63.1159x; 1.0102x over previous
import functools

import jax
import jax.numpy as jnp
import numpy as np
from jax import lax
from jax.experimental import pallas as pl
from jax.experimental.pallas import tpu as pltpu
from jax.experimental.pallas import tpu_sc as plsc

_B, _H, _W, _C = 2, 192, 192, 96
_P = _H * _W
_R = (_H * 2) * (_W * 2)
_HALF = _R // 2
_CHUNK = 4608
_NPAIRS = _B * _C
_NWORK = 32
_TASKS_PER_W = _NPAIRS // _NWORK

_THIRD = np.float32(1.0 / 3.0)


def _sc_unpool(mask_t, upd_t):
    mesh = plsc.VectorSubcoreMesh(core_axis_name="c", subcore_axis_name="s")

    @functools.partial(
        pl.kernel,
        mesh=mesh,
        out_type=jax.ShapeDtypeStruct((_NPAIRS, _R), jnp.float32),
        scratch_types=[
            pltpu.VMEM((_HALF,), jnp.float32),
            pltpu.VMEM((_P,), jnp.int32),
            pltpu.VMEM((2, _CHUNK), jnp.int32),
            pltpu.VMEM((2, _CHUNK), jnp.float32),
            pltpu.SemaphoreType.DMA,
            pltpu.SemaphoreType.DMA,
            pltpu.SemaphoreType.DMA,
        ],
        compiler_params=pltpu.CompilerParams(needs_layout_passes=False),
    )
    def k(mask_hbm, upd_hbm, out_hbm, acc, rcache, mbuf, ubuf, sem_a, sem_b,
          out_sem):
        wid = lax.axis_index("s") * 2 + lax.axis_index("c")
        sems = (sem_a, sem_b)
        nchunks = _P // _CHUNK
        zeros = jnp.zeros((16,), jnp.float32)

        def task(j, carry):
            pair = wid * _TASKS_PER_W + j

            def issue(ck, half):
                par = ck % 2
                off = ck * _CHUNK
                cu = pltpu.async_copy(
                    upd_hbm.at[pair, pl.ds(off, _CHUNK)], ubuf.at[par],
                    sems[par])
                if half == 0:
                    cm = pltpu.async_copy(
                        mask_hbm.at[pair, pl.ds(off, _CHUNK)], mbuf.at[par],
                        sems[par])
                    return cm, cu
                return (cu,)

            for half in (0, 1):
                lo = half * _HALF
                cps = {0: issue(0, half)}

                drain = pltpu.make_async_copy(
                    acc, out_hbm.at[pair, pl.ds(lo, _HALF)], out_sem)
                if half == 0:
                    @pl.when(j > 0)
                    def _():
                        drain.wait()
                else:
                    drain.wait()

                @plsc.parallel_loop(0, _HALF // 16, unroll=8)
                def _(i):
                    acc[pl.ds(i * 16, 16)] = zeros

                for ck in range(nchunks):
                    par = ck % 2
                    coff = ck * _CHUNK
                    if ck + 1 < nchunks:
                        cps[ck + 1] = issue(ck + 1, half)
                    for cp in cps.pop(ck):
                        cp.wait()

                    if half == 0:
                        @plsc.parallel_loop(0, _CHUNK // 16, unroll=8)
                        def _(g, par=par, coff=coff):
                            s = g * 16
                            m = mbuf[par, pl.ds(s, 16)]
                            v = ubuf[par, pl.ds(s, 16)]
                            t5 = lax.shift_right_logical(m, 5)
                            r = (t5.astype(jnp.float32) * _THIRD).astype(
                                jnp.int32)
                            rcache[pl.ds(coff + s, 16)] = r
                            ok = r < _HALF
                            plsc.addupdate_scatter(acc, [r], v, mask=ok)
                    else:
                        @plsc.parallel_loop(0, _CHUNK // 16, unroll=8)
                        def _(g, par=par, coff=coff):
                            s = g * 16
                            r = rcache[pl.ds(coff + s, 16)]
                            v = ubuf[par, pl.ds(s, 16)]
                            idx = r - _HALF
                            ok = r >= _HALF
                            plsc.addupdate_scatter(acc, [idx], v, mask=ok)

                pltpu.async_copy(
                    acc, out_hbm.at[pair, pl.ds(lo, _HALF)], out_sem)

            return carry

        lax.fori_loop(0, _TASKS_PER_W, task, 0)
        pltpu.make_async_copy(
            acc, out_hbm.at[0, pl.ds(0, _HALF)], out_sem).wait()

    return k(mask_t, upd_t)


def kernel(updates, mask):
    B, H, W, C = updates.shape
    Ho, Wo = H * 2, W * 2
    m32 = mask.astype(jnp.int32)
    u_t = updates.reshape(B, H * W, C).transpose(0, 2, 1).reshape(B * C, H * W)
    m_t = m32.reshape(B, H * W, C).transpose(0, 2, 1).reshape(B * C, H * W)
    out = _sc_unpool(m_t, u_t)
    return out.reshape(B, C, Ho * Wo).transpose(0, 2, 1).reshape(B, Ho, Wo, C)
